# Initial kernel scaffold; baseline (speedup 1.0000x reference)
import jax
import jax.numpy as jnp
from jax.experimental import pallas as pl


def kernel(x, code):
    raise NotImplementedError("write your pallas kernel here")


# speedup vs baseline: 11.8448x; 11.8448x over previous
import functools

import jax
import jax.numpy as jnp
from jax import lax
from jax.experimental import pallas as pl
from jax.experimental.pallas import tpu as pltpu

B = 2
N = 2048
CX = 384
CC = 96
K = 16
R = 256
RT = N // R


def _tc_kernel(xc_ref, cc_ref, xt_ref, ct_ref,
               a_ref, c_ref, clo_ref, far_ref, nn_ref,
               xn_scr, ccb_scr, sqx_scr, sqc_scr):
    rt = pl.program_id(1)

    @pl.when(rt == 0)
    def _init_cols():
        xcb = xc_ref[0]
        nrm2 = jnp.sum(xcb * xcb, axis=0, keepdims=True)
        inv = 1.0 / jnp.maximum(jnp.sqrt(nrm2), 1e-12)
        xn = xcb * inv
        sqx_scr[...] = jnp.sum(xn * xn, axis=0, keepdims=True)
        xn_scr[...] = xn.astype(jnp.bfloat16)
        ccb = cc_ref[0]
        sqc_scr[...] = jnp.sum(ccb * ccb, axis=0, keepdims=True)
        ccb_scr[...] = ccb.astype(jnp.bfloat16)
        a_ref[0] = jnp.zeros((1, N), jnp.float32)
        c_ref[0] = jnp.zeros((1, N), jnp.float32)

    xrb = xt_ref[0]
    rn2 = jnp.sum(xrb * xrb, axis=1, keepdims=True)
    xrn = xrb * (1.0 / jnp.maximum(jnp.sqrt(rn2), 1e-12))
    sqx_r = jnp.sum(xrn * xrn, axis=1, keepdims=True)
    crb = ct_ref[0]
    sqc_r = jnp.sum(crb * crb, axis=1, keepdims=True)

    dims = (((1,), (0,)), ((), ()))
    gx = lax.dot_general(xrn.astype(jnp.bfloat16), xn_scr[...], dims,
                         preferred_element_type=jnp.float32)
    gc = lax.dot_general(crb.astype(jnp.bfloat16), ccb_scr[...], dims,
                         preferred_element_type=jnp.float32)
    dx = jnp.maximum(sqx_r + sqx_scr[...] - 2.0 * gx, 0.0)
    dc = jnp.maximum(sqc_r + sqc_scr[...] - 2.0 * gc, 0.0)

    dcc = jnp.maximum(dc, 1e-5)
    a_ref[0] += jnp.sum(dcc * dx, axis=0, keepdims=True)
    c_ref[0] += jnp.sum(dcc, axis=0, keepdims=True)

    col_ids = lax.broadcasted_iota(jnp.int32, (R, N), 1)
    row_ids = rt * R + lax.broadcasted_iota(jnp.int32, (R, 1), 0)
    diag = col_ids == row_ids
    inf = jnp.float32(jnp.inf)
    clo = jnp.argmin(jnp.where(diag, inf, dx), axis=1)
    far = jnp.argmax(jnp.where(diag, -inf, dx), axis=1)
    clo_ref[...] = clo.reshape(1, 1, R, 1)
    far_ref[...] = far.reshape(1, 1, R, 1)

    neg_d = -(dx + dc)
    cols = []
    for _ in range(K):
        i = jnp.argmax(neg_d, axis=1)
        cols.append(i.reshape(R, 1))
        neg_d = jnp.where(col_ids == i.reshape(R, 1), -inf, neg_d)
    nn_ref[...] = jnp.concatenate(cols, axis=1).reshape(1, 1, R, K)


def _tc_call(xc, cc, xt, ct):
    grid = (B, RT)
    return pl.pallas_call(
        _tc_kernel,
        grid=grid,
        in_specs=[
            pl.BlockSpec((1, CX, N), lambda b, rt: (b, 0, 0)),
            pl.BlockSpec((1, CC, N), lambda b, rt: (b, 0, 0)),
            pl.BlockSpec((1, R, CX), lambda b, rt: (b, rt, 0)),
            pl.BlockSpec((1, R, CC), lambda b, rt: (b, rt, 0)),
        ],
        out_specs=[
            pl.BlockSpec((1, 1, N), lambda b, rt: (b, 0, 0)),
            pl.BlockSpec((1, 1, N), lambda b, rt: (b, 0, 0)),
            pl.BlockSpec((1, 1, R, 1), lambda b, rt: (b, rt, 0, 0)),
            pl.BlockSpec((1, 1, R, 1), lambda b, rt: (b, rt, 0, 0)),
            pl.BlockSpec((1, 1, R, K), lambda b, rt: (b, rt, 0, 0)),
        ],
        out_shape=[
            jax.ShapeDtypeStruct((B, 1, N), jnp.float32),
            jax.ShapeDtypeStruct((B, 1, N), jnp.float32),
            jax.ShapeDtypeStruct((B, RT, R, 1), jnp.int32),
            jax.ShapeDtypeStruct((B, RT, R, 1), jnp.int32),
            jax.ShapeDtypeStruct((B, RT, R, K), jnp.int32),
        ],
        scratch_shapes=[
            pltpu.VMEM((CX, N), jnp.bfloat16),
            pltpu.VMEM((CC, N), jnp.bfloat16),
            pltpu.VMEM((1, N), jnp.float32),
            pltpu.VMEM((1, N), jnp.float32),
        ],
    )(xc, cc, xt, ct)


def _finalize_loss(a, c, clo, far):
    s0 = jnp.sum(-a + 0.12 * c)
    s1 = jnp.sum(-jnp.take_along_axis(a, clo, 1)
                 + 0.2 * jnp.take_along_axis(c, clo, 1))
    s2 = jnp.sum(-jnp.take_along_axis(a, far, 1)
                 + 1.0 * jnp.take_along_axis(c, far, 1))
    return (0.1 * s0 + 1.0 * s1 + 0.15 * s2) / (B * N * N)


def kernel(x, code):
    xc = jnp.squeeze(x, -1)
    cc = jnp.squeeze(code, -1)
    xt = xc.transpose(0, 2, 1)
    ct = cc.transpose(0, 2, 1)
    a, c, clo, far, nn = _tc_call(xc, cc, xt, ct)
    a = a.reshape(B, N)
    c = c.reshape(B, N)
    clo = clo.reshape(B, N)
    far = far.reshape(B, N)
    loss = _finalize_loss(a, c, clo, far)
    nn_idx = nn.reshape(B, N, K)
    center = jnp.broadcast_to(
        jnp.arange(N, dtype=jnp.int32)[None, :, None], (B, N, K))
    edge_index = jnp.stack((nn_idx, center), axis=0)
    return edge_index, loss


# speedup vs baseline: 12.9184x; 1.0906x over previous
import functools

import jax
import jax.numpy as jnp
from jax import lax
from jax.experimental import pallas as pl
from jax.experimental.pallas import tpu as pltpu
from jax.experimental.pallas import tpu_sc as plsc

B = 2
N = 2048
CX = 384
CC = 96
K = 16
R = 256
RT = N // R


def _tc_kernel(xc_ref, cc_ref, xt_ref, ct_ref,
               a_ref, c_ref, clo_ref, far_ref, nn_ref,
               xn_scr, ccb_scr, sqx_scr, sqc_scr):
    rt = pl.program_id(1)

    @pl.when(rt == 0)
    def _init_cols():
        xcb = xc_ref[0]
        nrm2 = jnp.sum(xcb * xcb, axis=0, keepdims=True)
        inv = 1.0 / jnp.maximum(jnp.sqrt(nrm2), 1e-12)
        xn = xcb * inv
        sqx_scr[...] = jnp.sum(xn * xn, axis=0, keepdims=True)
        xn_scr[...] = xn.astype(jnp.bfloat16)
        ccb = cc_ref[0]
        sqc_scr[...] = jnp.sum(ccb * ccb, axis=0, keepdims=True)
        ccb_scr[...] = ccb.astype(jnp.bfloat16)
        a_ref[0] = jnp.zeros((1, N), jnp.float32)
        c_ref[0] = jnp.zeros((1, N), jnp.float32)

    xrb = xt_ref[0]
    rn2 = jnp.sum(xrb * xrb, axis=1, keepdims=True)
    xrn = xrb * (1.0 / jnp.maximum(jnp.sqrt(rn2), 1e-12))
    sqx_r = jnp.sum(xrn * xrn, axis=1, keepdims=True)
    crb = ct_ref[0]
    sqc_r = jnp.sum(crb * crb, axis=1, keepdims=True)

    dims = (((1,), (0,)), ((), ()))
    gx = lax.dot_general(xrn.astype(jnp.bfloat16), xn_scr[...], dims,
                         preferred_element_type=jnp.float32)
    gc = lax.dot_general(crb.astype(jnp.bfloat16), ccb_scr[...], dims,
                         preferred_element_type=jnp.float32)
    dx = jnp.maximum(sqx_r + sqx_scr[...] - 2.0 * gx, 0.0)
    dc = jnp.maximum(sqc_r + sqc_scr[...] - 2.0 * gc, 0.0)

    dcc = jnp.maximum(dc, 1e-5)
    a_ref[0] += jnp.sum(dcc * dx, axis=0, keepdims=True)
    c_ref[0] += jnp.sum(dcc, axis=0, keepdims=True)

    col_ids = lax.broadcasted_iota(jnp.int32, (R, N), 1)
    row_ids = rt * R + lax.broadcasted_iota(jnp.int32, (R, 1), 0)
    diag = col_ids == row_ids
    inf = jnp.float32(jnp.inf)
    clo = jnp.argmin(jnp.where(diag, inf, dx), axis=1)
    far = jnp.argmax(jnp.where(diag, -inf, dx), axis=1)
    clo_ref[...] = clo.reshape(1, 1, R, 1)
    far_ref[...] = far.reshape(1, 1, R, 1)

    neg_d = -(dx + dc)
    cols = []
    for _ in range(K):
        i = jnp.argmax(neg_d, axis=1)
        cols.append(i.reshape(R, 1))
        neg_d = jnp.where(col_ids == i.reshape(R, 1), -inf, neg_d)
    nn_ref[...] = jnp.concatenate(cols, axis=1).reshape(1, 1, R, K)


def _tc_call(xc, cc, xt, ct):
    grid = (B, RT)
    return pl.pallas_call(
        _tc_kernel,
        grid=grid,
        in_specs=[
            pl.BlockSpec((1, CX, N), lambda b, rt: (b, 0, 0)),
            pl.BlockSpec((1, CC, N), lambda b, rt: (b, 0, 0)),
            pl.BlockSpec((1, R, CX), lambda b, rt: (b, rt, 0)),
            pl.BlockSpec((1, R, CC), lambda b, rt: (b, rt, 0)),
        ],
        out_specs=[
            pl.BlockSpec((1, 1, N), lambda b, rt: (b, 0, 0)),
            pl.BlockSpec((1, 1, N), lambda b, rt: (b, 0, 0)),
            pl.BlockSpec((1, 1, R, 1), lambda b, rt: (b, rt, 0, 0)),
            pl.BlockSpec((1, 1, R, 1), lambda b, rt: (b, rt, 0, 0)),
            pl.BlockSpec((1, 1, R, K), lambda b, rt: (b, rt, 0, 0)),
        ],
        out_shape=[
            jax.ShapeDtypeStruct((B, 1, N), jnp.float32),
            jax.ShapeDtypeStruct((B, 1, N), jnp.float32),
            jax.ShapeDtypeStruct((B, RT, R, 1), jnp.int32),
            jax.ShapeDtypeStruct((B, RT, R, 1), jnp.int32),
            jax.ShapeDtypeStruct((B, RT, R, K), jnp.int32),
        ],
        scratch_shapes=[
            pltpu.VMEM((CX, N), jnp.bfloat16),
            pltpu.VMEM((CC, N), jnp.bfloat16),
            pltpu.VMEM((1, N), jnp.float32),
            pltpu.VMEM((1, N), jnp.float32),
        ],
    )(xc, cc, xt, ct)


NW = 32
JW = B * N // NW
L = 16


def _sc_kernel(a_hbm, c_hbm, clo_hbm, far_hbm, out_hbm,
               a_v, c_v, clo_v, far_v, res_v):
    wid = lax.axis_index("s") * 2 + lax.axis_index("c")
    pltpu.sync_copy(a_hbm, a_v)
    pltpu.sync_copy(c_hbm, c_v)
    pltpu.sync_copy(clo_hbm, clo_v)
    pltpu.sync_copy(far_hbm, far_v)
    acc0 = jnp.zeros((L,), jnp.float32)
    acc1 = jnp.zeros((L,), jnp.float32)
    acc2 = jnp.zeros((L,), jnp.float32)
    base = wid * (N // NW)
    for b in range(B):
        boff = jnp.full((L,), b * N, jnp.int32)
        for t in range(N // NW // L):
            sl = pl.ds(b * N + base + t * L, L)
            ci = clo_v[sl] + boff
            fi = far_v[sl] + boff
            av = plsc.load_gather(a_v, [ci])
            cv = plsc.load_gather(c_v, [ci])
            acc1 += 0.2 * cv - av
            av = plsc.load_gather(a_v, [fi])
            cv = plsc.load_gather(c_v, [fi])
            acc2 += cv - av
            acc0 += 0.12 * c_v[sl] - a_v[sl]
    res_v[pl.ds(0, L)] = acc0
    res_v[pl.ds(L, L)] = acc1
    res_v[pl.ds(2 * L, L)] = acc2
    pltpu.sync_copy(res_v, out_hbm.at[wid])


@functools.partial(
    pl.kernel,
    mesh=plsc.VectorSubcoreMesh(core_axis_name="c", subcore_axis_name="s"),
    out_type=jax.ShapeDtypeStruct((NW, 3 * L), jnp.float32),
    compiler_params=pltpu.CompilerParams(needs_layout_passes=False),
    scratch_types=[
        pltpu.VMEM((B * N,), jnp.float32),
        pltpu.VMEM((B * N,), jnp.float32),
        pltpu.VMEM((B * N,), jnp.int32),
        pltpu.VMEM((B * N,), jnp.int32),
        pltpu.VMEM((3 * L,), jnp.float32),
    ],
)
def _sc_call(a_hbm, c_hbm, clo_hbm, far_hbm, out_hbm,
             a_v, c_v, clo_v, far_v, res_v):
    _sc_kernel(a_hbm, c_hbm, clo_hbm, far_hbm, out_hbm,
               a_v, c_v, clo_v, far_v, res_v)


def _finalize_loss(a, c, clo, far):
    parts = _sc_call(a, c, clo, far)
    s0 = jnp.sum(parts[:, 0:L])
    s1 = jnp.sum(parts[:, L:2 * L])
    s2 = jnp.sum(parts[:, 2 * L:3 * L])
    return (0.1 * s0 + 1.0 * s1 + 0.15 * s2) / (B * N * N)


def kernel(x, code):
    xc = jnp.squeeze(x, -1)
    cc = jnp.squeeze(code, -1)
    xt = xc.transpose(0, 2, 1)
    ct = cc.transpose(0, 2, 1)
    a, c, clo, far, nn = _tc_call(xc, cc, xt, ct)
    loss = _finalize_loss(a.reshape(B * N), c.reshape(B * N),
                          clo.reshape(B * N), far.reshape(B * N))
    nn_idx = nn.reshape(B, N, K)
    center = jnp.broadcast_to(
        jnp.arange(N, dtype=jnp.int32)[None, :, None], (B, N, K))
    edge_index = jnp.stack((nn_idx, center), axis=0)
    return edge_index, loss
